# baseline (speedup 1.0000x reference)
import functools

import jax
import jax.numpy as jnp
from jax import lax
from jax.experimental import pallas as pl
from jax.experimental.pallas import tpu as pltpu
from jax.experimental.pallas import tpu_sc as plsc

BATCH = 16384
DF = 64

_NC = 2
_NS = 16
_NW = _NC * _NS
_BPW = BATCH // _NW
_CH = 128
_NCH = _BPW // _CH


@functools.cache
def _make_sc_gather():
    mesh = plsc.VectorSubcoreMesh(core_axis_name="c", subcore_axis_name="s")

    @functools.partial(
        pl.kernel,
        mesh=mesh,
        out_type=(
            jax.ShapeDtypeStruct((BATCH, DF), jnp.float32),
            jax.ShapeDtypeStruct((BATCH, DF), jnp.float32),
        ),
        scratch_types=[
            pltpu.VMEM((_NCH, _CH), jnp.int32),
            pltpu.VMEM((_NCH, _CH), jnp.int32),
            pltpu.VMEM((_BPW, DF), jnp.float32),
            pltpu.VMEM((_BPW, DF), jnp.float32),
            pltpu.SemaphoreType.DMA,
        ],
        compiler_params=pltpu.CompilerParams(use_tc_tiling_on_sc=False),
    )
    def _sc_gather(uid_hbm, sid_hbm, utab_hbm, stab_hbm, uout_hbm, sout_hbm,
                   uidx_v, sidx_v, urows_v, srows_v, sem):
        wid = lax.axis_index("s") * _NC + lax.axis_index("c")
        base = wid * _BPW
        pltpu.sync_copy(uid_hbm.at[pl.ds(wid * _NCH, _NCH)], uidx_v)
        pltpu.sync_copy(sid_hbm.at[pl.ds(wid * _NCH, _NCH)], sidx_v)
        copies = []
        for j in range(_NCH):
            copies.append(pltpu.async_copy(
                utab_hbm.at[uidx_v.at[j]], urows_v.at[pl.ds(j * _CH, _CH)], sem))
            copies.append(pltpu.async_copy(
                stab_hbm.at[sidx_v.at[j]], srows_v.at[pl.ds(j * _CH, _CH)], sem))
        for c in copies:
            c.wait()
        pltpu.sync_copy(urows_v, uout_hbm.at[pl.ds(base, _BPW)])
        pltpu.sync_copy(srows_v, sout_hbm.at[pl.ds(base, _BPW)])

    return _sc_gather


_BLK = 1024


def _mlp_body(u_ref, s_ref, w0u_ref, w0s_ref, b0_ref, w1_ref, b1_ref,
              w2_ref, b2_ref, w3_ref, b3_ref, out_ref):
    x = (jnp.dot(u_ref[...], w0u_ref[...], preferred_element_type=jnp.float32)
         + jnp.dot(s_ref[...], w0s_ref[...], preferred_element_type=jnp.float32)
         + b0_ref[...])
    x = jnp.maximum(x, 0.0)
    x = jnp.dot(x, w1_ref[...], preferred_element_type=jnp.float32) + b1_ref[...]
    x = jnp.maximum(x, 0.0)
    x = jnp.dot(x, w2_ref[...], preferred_element_type=jnp.float32) + b2_ref[...]
    x = jnp.maximum(x, 0.0)
    out_ref[...] = jnp.sum(x * w3_ref[...], axis=1) + b3_ref[0, 0]


def _mlp(u, s, w0u, w0s, b0, w1, b1, w2, b2, w3, b3):
    grid = (BATCH // _BLK,)
    full = lambda shape: pl.BlockSpec(shape, lambda i: (0,) * len(shape))
    return pl.pallas_call(
        _mlp_body,
        grid=grid,
        in_specs=[
            pl.BlockSpec((_BLK, DF), lambda i: (i, 0)),
            pl.BlockSpec((_BLK, DF), lambda i: (i, 0)),
            full((DF, 128)),
            full((DF, 128)),
            full((1, 128)),
            full((128, 64)),
            full((1, 64)),
            full((64, 32)),
            full((1, 32)),
            full((1, 32)),
            full((1, 1)),
        ],
        out_specs=pl.BlockSpec((_BLK,), lambda i: (i,)),
        out_shape=jax.ShapeDtypeStruct((BATCH,), jnp.float32),
    )(u, s, w0u, w0s, b0, w1, b1, w2, b2, w3, b3)


def kernel(user_ids, song_ids, user_table, song_table,
           W0, b0, W1, b1, W2, b2, W3, b3):
    uid = user_ids.astype(jnp.int32).reshape(_NW * _NCH, _CH)
    sid = song_ids.astype(jnp.int32).reshape(_NW * _NCH, _CH)
    u, s = _make_sc_gather()(uid, sid, user_table, song_table)
    w0t = W0.T
    return _mlp(
        u, s,
        w0t[:DF], w0t[DF:], b0.reshape(1, 128),
        W1.T, b1.reshape(1, 64),
        W2.T, b2.reshape(1, 32),
        W3, b3.reshape(1, 1),
    )
